# baseline (speedup 1.0000x reference)
import functools
import math

import jax
import jax.numpy as jnp
from jax import lax
from jax.experimental import pallas as pl
from jax.experimental.pallas import tpu as pltpu
from jax.experimental.pallas import tpu_sc as plsc

N = 100000
D = 128
B = 64

CHUNK = 128
FULL_STEPS = N // CHUNK
TAIL = N - FULL_STEPS * CHUNK
NW = 32
STEPS_BASE = FULL_STEPS // NW
STEPS_REM = FULL_STEPS % NW


def _seg_body(v_hbm, bk_hbm, sums_out, cnts_out,
              idx_v, v_rows, idx_t, v_t, acc, ones_w,
              sh_all, sh_all_c):
    c = lax.axis_index("c")
    s = lax.axis_index("s")
    w = s * 2 + c

    def _fill_zeros(i, carry):
        for jj in range(D // 16):
            acc[i, pl.ds(jj * 16, 16)] = jnp.zeros((16,), jnp.float32)
        return carry

    lax.fori_loop(0, B, _fill_zeros, 0)

    def _fill_ones(i, carry):
        for jj in range(D // 16):
            ones_w[i, pl.ds(jj * 16, 16)] = jnp.ones((16,), jnp.float32)
        return carry

    lax.fori_loop(0, CHUNK, _fill_ones, 0)

    pltpu.sync_copy(acc, sh_all.at[pl.ds(s * B, B)])
    pltpu.sync_copy(acc, sh_all_c.at[pl.ds(s * B, B)])

    n_steps = STEPS_BASE + jnp.where(w < STEPS_REM, 1, 0)
    row0 = s * B

    def _offset_idx(idx_ref, n):
        for k in range(n // 16):
            sl = pl.ds(k * 16, 16)
            idx_ref[sl] = idx_ref[sl] + row0

    def _step(j, carry):
        base = (w + j * NW) * CHUNK
        pltpu.sync_copy(bk_hbm.at[pl.ds(base, CHUNK)], idx_v)
        pltpu.sync_copy(v_hbm.at[pl.ds(base, CHUNK)], v_rows)
        _offset_idx(idx_v, CHUNK)
        pltpu.sync_copy(v_rows, sh_all.at[idx_v], add=True)
        pltpu.sync_copy(ones_w, sh_all_c.at[idx_v], add=True)
        return carry

    lax.fori_loop(0, n_steps, _step, 0)

    @pl.when(w == 0)
    def _tail():
        base = FULL_STEPS * CHUNK
        pltpu.sync_copy(bk_hbm.at[pl.ds(base, TAIL)], idx_t)
        pltpu.sync_copy(v_hbm.at[pl.ds(base, TAIL)], v_t)
        pltpu.sync_copy(v_t, sh_all.at[idx_t], add=True)
        pltpu.sync_copy(ones_w.at[pl.ds(0, TAIL)], sh_all_c.at[idx_t],
                        add=True)

    pltpu.sync_copy(sh_all.at[pl.ds(s * B, B)], acc)
    pltpu.sync_copy(acc, sums_out.at[pl.ds(w * B, B)])
    pltpu.sync_copy(sh_all_c.at[pl.ds(s * B, B)], acc)
    pltpu.sync_copy(acc, cnts_out.at[pl.ds(w * B, B)])


@functools.cache
def _seg_sums_sc():
    mesh = plsc.VectorSubcoreMesh(core_axis_name="c", subcore_axis_name="s")
    return pl.kernel(
        _seg_body,
        out_type=[
            jax.ShapeDtypeStruct((NW * B, D), jnp.float32),
            jax.ShapeDtypeStruct((NW * B, D), jnp.float32),
        ],
        mesh=mesh,
        scratch_types=[
            pltpu.VMEM((CHUNK,), jnp.int32),
            pltpu.VMEM((CHUNK, D), jnp.float32),
            pltpu.VMEM((TAIL,), jnp.int32),
            pltpu.VMEM((TAIL, D), jnp.float32),
            pltpu.VMEM((B, D), jnp.float32),
            pltpu.VMEM((CHUNK, D), jnp.float32),
            pltpu.VMEM_SHARED((16 * B, D), jnp.float32),
            pltpu.VMEM_SHARED((16 * B, D), jnp.float32),
        ],
    )


def _proto_body(sums_ref, cnts_ref, wr_ref, pw_ref):
    s3 = jnp.sum(sums_ref[...].reshape(NW, B, D), axis=0)
    cn = jnp.sum(cnts_ref[...].reshape(NW, B, D)[:, :, 0:1], axis=0)
    gmean = jnp.sum(s3, axis=0, keepdims=True) * (1.0 / N)
    protos = s3 / jnp.maximum(cn, 1.0)
    protos = jnp.where(cn == 0.0, gmean, protos)
    pw_ref[...] = lax.dot_general(protos, wr_ref[...], (((1,), (1,)), ((), ())),
                                  preferred_element_type=jnp.float32)


def _proto_stage(sums_p, cnts_p, W_r):
    return pl.pallas_call(
        _proto_body,
        out_shape=jax.ShapeDtypeStruct((B, D), jnp.float32),
    )(sums_p, cnts_p, W_r)


ROWS_BLK = 2048
GRID = (N + ROWS_BLK - 1) // ROWS_BLK

_INV_LOG_B = 1.0 / math.log(float(B))


def _dense_body(h_ref, lg_ref, pw_ref, g_ref, b_ref, out_ref, conf_ref):
    lg = lg_ref[...]
    m = jnp.max(lg, axis=-1, keepdims=True)
    e = jnp.exp(lg - m)
    p = e / jnp.sum(e, axis=-1, keepdims=True)
    ent = -jnp.sum(p * jnp.log(p + 1e-9), axis=-1, keepdims=True)
    gate = ent * _INV_LOG_B

    residual = jnp.dot(p, pw_ref[...], preferred_element_type=jnp.float32)
    h = h_ref[...] + gate * residual
    mean = jnp.mean(h, axis=-1, keepdims=True)
    var = jnp.mean((h - mean) ** 2, axis=-1, keepdims=True)
    out_ref[...] = (h - mean) * lax.rsqrt(var + 1e-5) * g_ref[...] + b_ref[...]
    conf_ref[...] = 1.0 - gate


def _dense_stage(h_fused, bucket_logits_q, pw, g2, b2):
    out, conf2 = pl.pallas_call(
        _dense_body,
        grid=(GRID,),
        in_specs=[
            pl.BlockSpec((ROWS_BLK, D), lambda i: (i, 0)),
            pl.BlockSpec((ROWS_BLK, B), lambda i: (i, 0)),
            pl.BlockSpec((B, D), lambda i: (0, 0)),
            pl.BlockSpec((1, D), lambda i: (0, 0)),
            pl.BlockSpec((1, D), lambda i: (0, 0)),
        ],
        out_specs=[
            pl.BlockSpec((ROWS_BLK, D), lambda i: (i, 0)),
            pl.BlockSpec((ROWS_BLK, 1), lambda i: (i, 0)),
        ],
        out_shape=[
            jax.ShapeDtypeStruct((N, D), jnp.float32),
            jax.ShapeDtypeStruct((N, 1), jnp.float32),
        ],
    )(h_fused, bucket_logits_q, pw, g2, b2)
    return out, conf2


def kernel(h_fused, V, bucket_logits_q, bk, W_r, ln_gamma, ln_beta):
    sums_p, cnts_p = _seg_sums_sc()(V, bk)
    pw = _proto_stage(sums_p, cnts_p, W_r)
    out, conf2 = _dense_stage(
        h_fused, bucket_logits_q, pw,
        ln_gamma.reshape(1, D), ln_beta.reshape(1, D))
    return out, conf2.reshape(N)


# speedup vs baseline: 1.0005x; 1.0005x over previous
import functools
import math

import jax
import jax.numpy as jnp
from jax import lax
from jax.experimental import pallas as pl
from jax.experimental.pallas import tpu as pltpu
from jax.experimental.pallas import tpu_sc as plsc

N = 100000
D = 128
B = 64

CHUNK = 128
FULL_STEPS = N // CHUNK
TAIL = N - FULL_STEPS * CHUNK
NW = 32
STEPS_BASE = FULL_STEPS // NW
STEPS_REM = FULL_STEPS % NW


def _seg_body(v_hbm, bk_hbm, sums_out, cnts_out,
              idx_v, v_rows, idx_t, v_t, acc, ones_w,
              sh_all, sh_all_c):
    c = lax.axis_index("c")
    s = lax.axis_index("s")
    w = s * 2 + c

    def _fill_zeros(i, carry):
        for jj in range(D // 16):
            acc[i, pl.ds(jj * 16, 16)] = jnp.zeros((16,), jnp.float32)
        return carry

    lax.fori_loop(0, B, _fill_zeros, 0)

    def _fill_ones(i, carry):
        for jj in range(D // 16):
            ones_w[i, pl.ds(jj * 16, 16)] = jnp.ones((16,), jnp.float32)
        return carry

    lax.fori_loop(0, CHUNK, _fill_ones, 0)

    pltpu.sync_copy(acc, sh_all.at[pl.ds(s * B, B)])
    pltpu.sync_copy(acc, sh_all_c.at[pl.ds(s * B, B)])

    n_steps = STEPS_BASE + jnp.where(w < STEPS_REM, 1, 0)
    row0 = s * B

    def _offset_idx(idx_ref, n):
        for k in range(n // 16):
            sl = pl.ds(k * 16, 16)
            idx_ref[sl] = idx_ref[sl] + row0

    def _step(j, carry):
        base = (w + j * NW) * CHUNK
        pltpu.sync_copy(bk_hbm.at[pl.ds(base, CHUNK)], idx_v)
        pltpu.sync_copy(v_hbm.at[pl.ds(base, CHUNK)], v_rows)
        _offset_idx(idx_v, CHUNK)
        pltpu.sync_copy(v_rows, sh_all.at[idx_v], add=True)
        pltpu.sync_copy(ones_w, sh_all_c.at[idx_v], add=True)
        return carry

    lax.fori_loop(0, n_steps, _step, 0)

    @pl.when(w == 0)
    def _tail():
        base = FULL_STEPS * CHUNK
        pltpu.sync_copy(bk_hbm.at[pl.ds(base, TAIL)], idx_t)
        pltpu.sync_copy(v_hbm.at[pl.ds(base, TAIL)], v_t)
        pltpu.sync_copy(v_t, sh_all.at[idx_t], add=True)
        pltpu.sync_copy(ones_w.at[pl.ds(0, TAIL)], sh_all_c.at[idx_t],
                        add=True)

    pltpu.sync_copy(sh_all.at[pl.ds(s * B, B)], acc)
    pltpu.sync_copy(acc, sums_out.at[pl.ds(w * B, B)])
    pltpu.sync_copy(sh_all_c.at[pl.ds(s * B, B)], acc)
    pltpu.sync_copy(acc, cnts_out.at[pl.ds(w * B, B)])


@functools.cache
def _seg_sums_sc():
    mesh = plsc.VectorSubcoreMesh(core_axis_name="c", subcore_axis_name="s")
    return pl.kernel(
        _seg_body,
        out_type=[
            jax.ShapeDtypeStruct((NW * B, D), jnp.float32),
            jax.ShapeDtypeStruct((NW * B, D), jnp.float32),
        ],
        mesh=mesh,
        scratch_types=[
            pltpu.VMEM((CHUNK,), jnp.int32),
            pltpu.VMEM((CHUNK, D), jnp.float32),
            pltpu.VMEM((TAIL,), jnp.int32),
            pltpu.VMEM((TAIL, D), jnp.float32),
            pltpu.VMEM((B, D), jnp.float32),
            pltpu.VMEM((CHUNK, D), jnp.float32),
            pltpu.VMEM_SHARED((16 * B, D), jnp.float32),
            pltpu.VMEM_SHARED((16 * B, D), jnp.float32),
        ],
    )


def _proto_body(sums_ref, cnts_ref, wr_ref, pw_ref):
    s3 = jnp.sum(sums_ref[...].reshape(NW, B, D), axis=0)
    cn = jnp.sum(cnts_ref[...].reshape(NW, B, D)[:, :, 0:1], axis=0)
    gmean = jnp.sum(s3, axis=0, keepdims=True) * (1.0 / N)
    protos = s3 / jnp.maximum(cn, 1.0)
    protos = jnp.where(cn == 0.0, gmean, protos)
    pw_ref[...] = lax.dot_general(protos, wr_ref[...], (((1,), (1,)), ((), ())),
                                  preferred_element_type=jnp.float32)


def _proto_stage(sums_p, cnts_p, W_r):
    return pl.pallas_call(
        _proto_body,
        out_shape=jax.ShapeDtypeStruct((B, D), jnp.float32),
    )(sums_p, cnts_p, W_r)


ROWS_BLK = 2048
GRID = (N + ROWS_BLK - 1) // ROWS_BLK

_INV_LOG_B = 1.0 / math.log(float(B))


def _dense_body(h_ref, lg_ref, pw_ref, g_ref, b_ref, out_ref, conf_ref):
    lg = lg_ref[...]
    m = jnp.max(lg, axis=-1, keepdims=True)
    e = jnp.exp(lg - m)
    p = e / jnp.sum(e, axis=-1, keepdims=True)
    ent = -jnp.sum(p * jnp.log(p + 1e-9), axis=-1, keepdims=True)
    gate = ent * _INV_LOG_B

    residual = jnp.dot(p, pw_ref[...], preferred_element_type=jnp.float32)
    h = h_ref[...] + gate * residual
    mean = jnp.mean(h, axis=-1, keepdims=True)
    var = jnp.mean((h - mean) ** 2, axis=-1, keepdims=True)
    out_ref[...] = (h - mean) * lax.rsqrt(var + 1e-5) * g_ref[...] + b_ref[...]
    conf_ref[...] = 1.0 - gate


def _dense_stage(h_fused, bucket_logits_q, pw, g2, b2):
    out, conf2 = pl.pallas_call(
        _dense_body,
        grid=(GRID,),
        in_specs=[
            pl.BlockSpec((ROWS_BLK, D), lambda i: (i, 0)),
            pl.BlockSpec((ROWS_BLK, B), lambda i: (i, 0)),
            pl.BlockSpec((B, D), lambda i: (0, 0)),
            pl.BlockSpec((1, D), lambda i: (0, 0)),
            pl.BlockSpec((1, D), lambda i: (0, 0)),
        ],
        out_specs=[
            pl.BlockSpec((ROWS_BLK, D), lambda i: (i, 0)),
            pl.BlockSpec((ROWS_BLK, 1), lambda i: (i, 0)),
        ],
        out_shape=[
            jax.ShapeDtypeStruct((N, D), jnp.float32),
            jax.ShapeDtypeStruct((N, 1), jnp.float32),
        ],
        compiler_params=pltpu.CompilerParams(
            dimension_semantics=("parallel",)),
    )(h_fused, bucket_logits_q, pw, g2, b2)
    return out, conf2


def kernel(h_fused, V, bucket_logits_q, bk, W_r, ln_gamma, ln_beta):
    sums_p, cnts_p = _seg_sums_sc()(V, bk)
    pw = _proto_stage(sums_p, cnts_p, W_r)
    out, conf2 = _dense_stage(
        h_fused, bucket_logits_q, pw,
        ln_gamma.reshape(1, D), ln_beta.reshape(1, D))
    return out, conf2.reshape(N)
